# baseline (speedup 1.0000x reference)
import jax
import jax.numpy as jnp
from jax.experimental import pallas as pl
from jax.experimental.pallas import tpu as pltpu
from jax.experimental.pallas import tpu_sc as plsc

_S = 576
_T = 32768
_TOPK = 1
_TGT = 577
_NPAD = 640
_D = 1280
_SENTINEL = 2_000_000

_NW = 32
_RPW = 8
_SC_ROWS = _NW * _RPW
_TC_ROWS = _S - _SC_ROWS
_TC_BLK = 64
_NV = _T // 16
_UNROLL = 8


def _tc_stage1_body(x_ref, vals_ref, idx_ref):
    x = x_ref[...]
    m = jnp.max(x, axis=1, keepdims=True)
    col = jax.lax.broadcasted_iota(jnp.int32, x.shape, 1)
    am = jnp.min(jnp.where(x == m, col, _T), axis=1, keepdims=True)
    vals_ref[...] = m
    idx_ref[...] = am


def _sc_stage1_body(attn_ref, vals_ref, idx_ref, buf, vvals, vidx, sem0, sem1):
    wid = jax.lax.axis_index("s") * 2 + jax.lax.axis_index("c")
    row0 = _TC_ROWS + wid * _RPW
    lane = jax.lax.broadcasted_iota(jnp.int32, (16,), 0)
    sems = (sem0, sem1)
    copies = [pltpu.async_copy(attn_ref.at[row0], buf.at[0], sem0), None]
    for k in range(_RPW):
        cur = k % 2
        if k + 1 < _RPW:
            nxt = (k + 1) % 2
            copies[nxt] = pltpu.async_copy(
                attn_ref.at[row0 + k + 1], buf.at[nxt], sems[nxt])
        copies[cur].wait()
        bk = buf.at[cur]

        def body(i, carry):
            bvs, bcs = carry
            new_bvs, new_bcs = [], []
            for u in range(_UNROLL):
                v = bk[pl.ds((i * _UNROLL + u) * 16, 16)]
                gt = v > bvs[u]
                new_bvs.append(jnp.where(gt, v, bvs[u]))
                new_bcs.append(jnp.where(gt, i, bcs[u]))
            return tuple(new_bvs), tuple(new_bcs)

        init = (tuple(jnp.full((16,), -jnp.inf, jnp.float32)
                      for _ in range(_UNROLL)),
                tuple(jnp.zeros((16,), jnp.int32) for _ in range(_UNROLL)))
        bvs, bcs = jax.lax.fori_loop(0, _NV // _UNROLL, body, init)

        acc_v = bvs[0]
        acc_i = bcs[0] * (_UNROLL * 16) + lane
        for u in range(1, _UNROLL):
            ai = bcs[u] * (_UNROLL * 16) + u * 16 + lane
            better = (bvs[u] > acc_v) | ((bvs[u] == acc_v) & (ai < acc_i))
            acc_v = jnp.where(better, bvs[u], acc_v)
            acc_i = jnp.where(better, ai, acc_i)
        vvals[k] = acc_v
        vidx[k] = acc_i
    pltpu.sync_copy(vvals, vals_ref.at[pl.ds(wid * _RPW, _RPW)])
    pltpu.sync_copy(vidx, idx_ref.at[pl.ds(wid * _RPW, _RPW)])


_sc_stage1 = pl.kernel(
    _sc_stage1_body,
    out_type=[jax.ShapeDtypeStruct((_SC_ROWS, 16), jnp.float32),
              jax.ShapeDtypeStruct((_SC_ROWS, 16), jnp.int32)],
    mesh=plsc.VectorSubcoreMesh(core_axis_name="c", subcore_axis_name="s"),
    scratch_types=[pltpu.VMEM((2, _T), jnp.float32),
                   pltpu.VMEM((_RPW, 16), jnp.float32),
                   pltpu.VMEM((_RPW, 16), jnp.int32),
                   pltpu.SemaphoreType.DMA,
                   pltpu.SemaphoreType.DMA],
)


def _lane_reduce_body(v_ref, i_ref, vals_ref, idx_ref):
    v = v_ref[...]
    ix = i_ref[...]
    m = jnp.max(v, axis=1, keepdims=True)
    am = jnp.min(jnp.where(v == m, ix, _T), axis=1, keepdims=True)
    vals_ref[...] = m
    idx_ref[...] = am


def _stage2_body(q_col_ref, q_row_ref, out_ref):
    q_col = q_col_ref[...]
    q_row = q_row_ref[...]
    s_col = jax.lax.broadcasted_iota(jnp.int32, (_NPAD, 1), 0)
    s_row = jax.lax.broadcasted_iota(jnp.int32, (1, _NPAD), 1)
    valid_col = s_col < _TGT
    valid_row = s_row < _TGT

    eq = (q_col == q_row).astype(jnp.int32)
    lt_ct = (s_col < s_row).astype(jnp.int32)
    lt_tc = (s_row < s_col).astype(jnp.int32)
    u_row = ((jnp.sum(eq * lt_ct, axis=0, keepdims=True) == 0)
             & valid_row).astype(jnp.int32)
    u_col = ((jnp.sum(eq * lt_tc, axis=1, keepdims=True) == 0)
             & valid_col).astype(jnp.int32)

    p_cnt = jnp.sum(u_row)
    k_pad = _TGT - p_cnt

    i_col = jax.lax.broadcasted_iota(jnp.int32, (_D, 1), 0)
    le = (q_row <= i_col).astype(jnp.int32)
    oc = jnp.sum(le * u_row, axis=1, keepdims=True)
    cs = oc + jnp.minimum(k_pad, (i_col + 1) - oc)
    cs_d = jnp.sum(u_row * (q_row < _D)) + k_pad

    j_row = s_row
    out_small = jnp.sum((cs <= j_row).astype(jnp.int32), axis=0,
                        keepdims=True)

    b_row = u_row * (q_row >= _D).astype(jnp.int32)
    b_col = u_col * (q_col >= _D).astype(jnp.int32)
    r_col = cs_d + jnp.sum((q_row < q_col).astype(jnp.int32) * b_row,
                           axis=1, keepdims=True)
    hit = (r_col == j_row).astype(jnp.int32) * b_col
    out_big = jnp.sum(hit * q_col, axis=0, keepdims=True)

    out_ref[...] = jnp.where(j_row < cs_d, out_small, out_big)


def _run(attn2d, target_num, top_k):
    n_blk = _TC_ROWS // _TC_BLK
    vals_tc, idx_tc = pl.pallas_call(
        _tc_stage1_body,
        grid=(n_blk,),
        in_specs=[pl.BlockSpec((_TC_BLK, _T), lambda i: (i, 0))],
        out_specs=[pl.BlockSpec((_TC_BLK, 1), lambda i: (i, 0)),
                   pl.BlockSpec((_TC_BLK, 1), lambda i: (i, 0))],
        out_shape=[jax.ShapeDtypeStruct((_TC_ROWS, 1), jnp.float32),
                   jax.ShapeDtypeStruct((_TC_ROWS, 1), jnp.int32)],
    )(attn2d)
    vals_sc16, idx_sc16 = _sc_stage1(attn2d)
    vals_sc, idx_sc = pl.pallas_call(
        _lane_reduce_body,
        out_shape=[jax.ShapeDtypeStruct((_SC_ROWS, 1), jnp.float32),
                   jax.ShapeDtypeStruct((_SC_ROWS, 1), jnp.int32)],
    )(vals_sc16, idx_sc16)

    vals = jnp.concatenate([vals_tc.reshape(_TC_ROWS),
                            vals_sc.reshape(_SC_ROWS)])
    idx = jnp.concatenate([idx_tc.reshape(_TC_ROWS),
                           idx_sc.reshape(_SC_ROWS)])
    shift = 1 + (jnp.asarray(top_k, jnp.int32) - _TOPK)
    q = jnp.concatenate([
        idx + shift,
        jnp.zeros((1,), jnp.int32),
        jnp.full((_NPAD - _TGT,), _SENTINEL, jnp.int32),
    ])
    picked_pad = pl.pallas_call(
        _stage2_body,
        out_shape=jax.ShapeDtypeStruct((1, _NPAD), jnp.int32),
    )(q.reshape(_NPAD, 1), q.reshape(1, _NPAD))
    picked = (picked_pad.reshape(_NPAD)[:_TGT]
              + (jnp.asarray(target_num, jnp.int32) - _TGT))
    return vals.reshape(1, _S, _TOPK), picked


def kernel(attn_qk, target_num, top_k):
    if attn_qk.ndim == 2:
        attn_qk = attn_qk[None]
    return _run(attn_qk.reshape(_S, _T), target_num, top_k)


# speedup vs baseline: 1.0699x; 1.0699x over previous
import functools

import jax
import jax.numpy as jnp
from jax.experimental import pallas as pl
from jax.experimental.pallas import tpu as pltpu
from jax.experimental.pallas import tpu_sc as plsc

_S = 576
_T = 32768
_TOPK = 1
_TGT = 577
_NPAD = 640
_D = 1280
_SENTINEL = 2_000_000

_NW = 32
_RPW = 8
_SC_ROWS = _NW * _RPW
_TC_ROWS = _S - _SC_ROWS
_TC_BLK = 64
_NV = _T // 16
_UNROLL = 8


def _tc_stage1_body(x_ref, vals_ref, idx_ref):
    x = x_ref[...]
    m = jnp.max(x, axis=1, keepdims=True)
    col = jax.lax.broadcasted_iota(jnp.int32, x.shape, 1)
    am = jnp.min(jnp.where(x == m, col, _T), axis=1, keepdims=True)
    vals_ref[...] = m
    idx_ref[...] = am


def _sc_stage1_body(attn_ref, vals_ref, idx_ref, buf, vvals, vidx, sem0, sem1):
    wid = jax.lax.axis_index("s") * 2 + jax.lax.axis_index("c")
    row0 = _TC_ROWS + wid * _RPW
    lane = jax.lax.broadcasted_iota(jnp.int32, (16,), 0)
    sems = (sem0, sem1)
    copies = [pltpu.async_copy(attn_ref.at[row0], buf.at[0], sem0), None]
    for k in range(_RPW):
        cur = k % 2
        if k + 1 < _RPW:
            nxt = (k + 1) % 2
            copies[nxt] = pltpu.async_copy(
                attn_ref.at[row0 + k + 1], buf.at[nxt], sems[nxt])
        copies[cur].wait()
        bk = buf.at[cur]

        def body(i, carry):
            bvs, bcs = carry
            new_bvs, new_bcs = [], []
            for u in range(_UNROLL):
                v = bk[pl.ds((i * _UNROLL + u) * 16, 16)]
                gt = v > bvs[u]
                new_bvs.append(jnp.where(gt, v, bvs[u]))
                new_bcs.append(jnp.where(gt, i, bcs[u]))
            return tuple(new_bvs), tuple(new_bcs)

        init = (tuple(jnp.full((16,), -jnp.inf, jnp.float32)
                      for _ in range(_UNROLL)),
                tuple(jnp.zeros((16,), jnp.int32) for _ in range(_UNROLL)))
        bvs, bcs = jax.lax.fori_loop(0, _NV // _UNROLL, body, init)

        acc_v = bvs[0]
        acc_i = bcs[0] * (_UNROLL * 16) + lane
        for u in range(1, _UNROLL):
            ai = bcs[u] * (_UNROLL * 16) + u * 16 + lane
            better = (bvs[u] > acc_v) | ((bvs[u] == acc_v) & (ai < acc_i))
            acc_v = jnp.where(better, bvs[u], acc_v)
            acc_i = jnp.where(better, ai, acc_i)
        vvals[k] = acc_v
        vidx[k] = acc_i
    pltpu.sync_copy(vvals, vals_ref.at[pl.ds(wid * _RPW, _RPW)])
    pltpu.sync_copy(vidx, idx_ref.at[pl.ds(wid * _RPW, _RPW)])


@functools.lru_cache(maxsize=1)
def _sc_stage1():
    return pl.kernel(
        _sc_stage1_body,
        out_type=[jax.ShapeDtypeStruct((_SC_ROWS, 16), jnp.float32),
                  jax.ShapeDtypeStruct((_SC_ROWS, 16), jnp.int32)],
        mesh=plsc.VectorSubcoreMesh(core_axis_name="c",
                                    subcore_axis_name="s"),
        scratch_types=[pltpu.VMEM((2, _T), jnp.float32),
                       pltpu.VMEM((_RPW, 16), jnp.float32),
                       pltpu.VMEM((_RPW, 16), jnp.int32),
                       pltpu.SemaphoreType.DMA,
                       pltpu.SemaphoreType.DMA],
    )


def _stage2_body(vals_tc_ref, idx_tc_ref, scv_ref, sci_ref, topk_ref,
                 tgt_ref, out_ref, vals_ref):
    scv = scv_ref[...]
    sci = sci_ref[...]
    scm = jnp.max(scv, axis=1, keepdims=True)
    scam = jnp.min(jnp.where(scv == scm, sci, _T), axis=1, keepdims=True)
    vals_ref[...] = jnp.concatenate([vals_tc_ref[...], scm], axis=0)

    shift = 1 + (topk_ref[0, 0] - _TOPK)
    tail = jax.lax.broadcasted_iota(jnp.int32, (_NPAD - _S, 1), 0)
    tail = jnp.where(tail == 0, 0, _SENTINEL)
    q_col = jnp.concatenate(
        [idx_tc_ref[...] + shift, scam + shift, tail], axis=0)
    s_col = jax.lax.broadcasted_iota(jnp.int32, (_NPAD, 1), 0)
    s_row = jax.lax.broadcasted_iota(jnp.int32, (1, _NPAD), 1)
    eye = (s_col == s_row).astype(jnp.float32)
    q_row = jax.lax.dot_general(
        q_col.astype(jnp.float32), eye, (((0,), (0,)), ((), ())),
        precision=jax.lax.Precision.HIGHEST,
        preferred_element_type=jnp.float32).astype(jnp.int32)
    valid_col = s_col < _TGT
    valid_row = s_row < _TGT

    eq = (q_col == q_row).astype(jnp.int32)
    lt_ct = (s_col < s_row).astype(jnp.int32)
    lt_tc = (s_row < s_col).astype(jnp.int32)
    u_row = ((jnp.sum(eq * lt_ct, axis=0, keepdims=True) == 0)
             & valid_row).astype(jnp.int32)
    u_col = ((jnp.sum(eq * lt_tc, axis=1, keepdims=True) == 0)
             & valid_col).astype(jnp.int32)

    p_cnt = jnp.sum(u_row)
    k_pad = _TGT - p_cnt

    i_col = jax.lax.broadcasted_iota(jnp.int32, (_D, 1), 0)
    le = (q_row <= i_col).astype(jnp.int32)
    oc = jnp.sum(le * u_row, axis=1, keepdims=True)
    cs = oc + jnp.minimum(k_pad, (i_col + 1) - oc)
    cs_d = jnp.sum(u_row * (q_row < _D)) + k_pad

    j_row = s_row
    out_small = jnp.sum((cs <= j_row).astype(jnp.int32), axis=0,
                        keepdims=True)

    b_row = u_row * (q_row >= _D).astype(jnp.int32)
    b_col = u_col * (q_col >= _D).astype(jnp.int32)
    r_col = cs_d + jnp.sum((q_row < q_col).astype(jnp.int32) * b_row,
                           axis=1, keepdims=True)
    hit = (r_col == j_row).astype(jnp.int32) * b_col
    out_big = jnp.sum(hit * q_col, axis=0, keepdims=True)

    out_ref[...] = jnp.where(j_row < cs_d, out_small, out_big)


def _run(attn2d, target_num, top_k):
    n_blk = _TC_ROWS // _TC_BLK
    vals_tc, idx_tc = pl.pallas_call(
        _tc_stage1_body,
        grid=(n_blk,),
        in_specs=[pl.BlockSpec((_TC_BLK, _T), lambda i: (i, 0))],
        out_specs=[pl.BlockSpec((_TC_BLK, 1), lambda i: (i, 0)),
                   pl.BlockSpec((_TC_BLK, 1), lambda i: (i, 0))],
        out_shape=[jax.ShapeDtypeStruct((_TC_ROWS, 1), jnp.float32),
                   jax.ShapeDtypeStruct((_TC_ROWS, 1), jnp.int32)],
    )(attn2d)
    vals_sc16, idx_sc16 = _sc_stage1()(attn2d)

    topk_arr = jnp.asarray(top_k, jnp.int32).reshape(1, 1)
    tgt_arr = jnp.asarray(target_num, jnp.int32).reshape(1, 1)
    picked_pad, vals = pl.pallas_call(
        _stage2_body,
        out_shape=[jax.ShapeDtypeStruct((1, _NPAD), jnp.int32),
                   jax.ShapeDtypeStruct((_S, 1), jnp.float32)],
    )(vals_tc, idx_tc, vals_sc16, idx_sc16, topk_arr, tgt_arr)
    picked = picked_pad.reshape(_NPAD)[:_TGT]
    return vals.reshape(1, _S, _TOPK), picked


def kernel(attn_qk, target_num, top_k):
    if attn_qk.ndim == 2:
        attn_qk = attn_qk[None]
    return _run(attn_qk.reshape(_S, _T), target_num, top_k)


# speedup vs baseline: 1.5789x; 1.4756x over previous
import jax
import jax.numpy as jnp
from jax.experimental import pallas as pl

_S = 576
_T = 32768
_TOPK = 1
_TGT = 577
_NPAD = 640
_D = 1280
_SENTINEL = 2_000_000
_BLK = 96


def _stage1_body(x_ref, vals_ref, idx_ref):
    x = x_ref[...]
    m = jnp.max(x, axis=1, keepdims=True)
    col = jax.lax.broadcasted_iota(jnp.int32, x.shape, 1)
    am = jnp.min(jnp.where(x == m, col, _T), axis=1, keepdims=True)
    vals_ref[...] = m[None]
    idx_ref[...] = am


def _stage2_body(idx_ref, topk_ref, tgt_ref, out_ref):
    shift = 1 + (topk_ref[0, 0] - _TOPK)
    tail = jax.lax.broadcasted_iota(jnp.int32, (_NPAD - _S, 1), 0)
    tail = jnp.where(tail == 0, 0, _SENTINEL)
    q_col = jnp.concatenate([idx_ref[...] + shift, tail], axis=0)
    s_col = jax.lax.broadcasted_iota(jnp.int32, (_NPAD, 1), 0)
    s_row = jax.lax.broadcasted_iota(jnp.int32, (1, _NPAD), 1)
    eye = (s_col == s_row).astype(jnp.float32)
    q_row = jax.lax.dot_general(
        q_col.astype(jnp.float32), eye, (((0,), (0,)), ((), ())),
        precision=jax.lax.Precision.HIGHEST,
        preferred_element_type=jnp.float32).astype(jnp.int32)
    valid_col = s_col < _TGT
    valid_row = s_row < _TGT

    eq = (q_col == q_row).astype(jnp.int32)
    lt_ct = (s_col < s_row).astype(jnp.int32)
    lt_tc = (s_row < s_col).astype(jnp.int32)
    u_row = ((jnp.sum(eq * lt_ct, axis=0, keepdims=True) == 0)
             & valid_row).astype(jnp.int32)
    u_col = ((jnp.sum(eq * lt_tc, axis=1, keepdims=True) == 0)
             & valid_col).astype(jnp.int32)

    p_cnt = jnp.sum(u_row)
    k_pad = _TGT - p_cnt

    i_col = jax.lax.broadcasted_iota(jnp.int32, (_D, 1), 0)
    le = (q_row <= i_col).astype(jnp.int32)
    oc = jnp.sum(le * u_row, axis=1, keepdims=True)
    cs = oc + jnp.minimum(k_pad, (i_col + 1) - oc)
    cs_d = jnp.sum(u_row * (q_row < _D)) + k_pad

    j_row = s_row
    out_small = jnp.sum((cs <= j_row).astype(jnp.int32), axis=0,
                        keepdims=True)

    b_row = u_row * (q_row >= _D).astype(jnp.int32)
    b_col = u_col * (q_col >= _D).astype(jnp.int32)
    r_col = cs_d + jnp.sum((q_row < q_col).astype(jnp.int32) * b_row,
                           axis=1, keepdims=True)
    hit = (r_col == j_row).astype(jnp.int32) * b_col
    out_big = jnp.sum(hit * q_col, axis=0, keepdims=True)

    picked = (jnp.where(j_row < cs_d, out_small, out_big)
              + (tgt_ref[0, 0] - _TGT))
    out_ref[...] = picked[:, :_TGT]


def _run(attn2d, target_num, top_k):
    vals, idx = pl.pallas_call(
        _stage1_body,
        grid=(_S // _BLK,),
        in_specs=[pl.BlockSpec((_BLK, _T), lambda i: (i, 0))],
        out_specs=[pl.BlockSpec((1, _BLK, 1), lambda i: (0, i, 0)),
                   pl.BlockSpec((_BLK, 1), lambda i: (i, 0))],
        out_shape=[jax.ShapeDtypeStruct((1, _S, 1), jnp.float32),
                   jax.ShapeDtypeStruct((_S, 1), jnp.int32)],
    )(attn2d)

    topk_arr = jnp.asarray(top_k, jnp.int32).reshape(1, 1)
    tgt_arr = jnp.asarray(target_num, jnp.int32).reshape(1, 1)
    picked = pl.pallas_call(
        _stage2_body,
        out_shape=jax.ShapeDtypeStruct((1, _TGT), jnp.int32),
    )(idx, topk_arr, tgt_arr)
    return vals, picked.reshape(_TGT)


def kernel(attn_qk, target_num, top_k):
    if attn_qk.ndim == 2:
        attn_qk = attn_qk[None]
    return _run(attn_qk.reshape(_S, _T), target_num, top_k)


# speedup vs baseline: 1.7025x; 1.0783x over previous
import jax
import jax.numpy as jnp
from jax.experimental import pallas as pl

_S = 576
_T = 32768
_TOPK = 1
_TGT = 577
_NPAD = 640
_D = 1280
_SENTINEL = 2_000_000
_BLK = 96


def _stage1_body(x_ref, vals_ref, idx_ref):
    x = x_ref[...]
    m = jnp.max(x, axis=1, keepdims=True)
    col = jax.lax.broadcasted_iota(jnp.int32, x.shape, 1)
    am = jnp.min(jnp.where(x == m, col, _T), axis=1, keepdims=True)
    vals_ref[...] = m
    idx_ref[...] = am


def _stage2_body(idx_ref, vals_ref, topk_ref, tgt_ref, out_ref, vrow_ref):
    sv_col = jax.lax.broadcasted_iota(jnp.int32, (_S, 1), 0)
    sv_row = jax.lax.broadcasted_iota(jnp.int32, (1, _S), 1)
    eye_s = (sv_col == sv_row).astype(jnp.float32)
    vrow_ref[...] = jax.lax.dot_general(
        vals_ref[...], eye_s, (((0,), (0,)), ((), ())),
        precision=jax.lax.Precision.HIGHEST,
        preferred_element_type=jnp.float32)
    shift = 1 + (topk_ref[0, 0] - _TOPK)
    tail = jax.lax.broadcasted_iota(jnp.int32, (_NPAD - _S, 1), 0)
    tail = jnp.where(tail == 0, 0, _SENTINEL)
    q_col = jnp.concatenate([idx_ref[...] + shift, tail], axis=0)
    s_col = jax.lax.broadcasted_iota(jnp.int32, (_NPAD, 1), 0)
    s_row = jax.lax.broadcasted_iota(jnp.int32, (1, _NPAD), 1)
    eye = (s_col == s_row).astype(jnp.float32)
    q_row = jax.lax.dot_general(
        q_col.astype(jnp.float32), eye, (((0,), (0,)), ((), ())),
        precision=jax.lax.Precision.HIGHEST,
        preferred_element_type=jnp.float32).astype(jnp.int32)
    valid_col = s_col < _TGT
    valid_row = s_row < _TGT

    eq = (q_col == q_row).astype(jnp.int32)
    lt_ct = (s_col < s_row).astype(jnp.int32)
    lt_tc = (s_row < s_col).astype(jnp.int32)
    u_row = ((jnp.sum(eq * lt_ct, axis=0, keepdims=True) == 0)
             & valid_row).astype(jnp.int32)
    u_col = ((jnp.sum(eq * lt_tc, axis=1, keepdims=True) == 0)
             & valid_col).astype(jnp.int32)

    p_cnt = jnp.sum(u_row)
    k_pad = _TGT - p_cnt

    i_col = jax.lax.broadcasted_iota(jnp.int32, (_D, 1), 0)
    le = (q_row <= i_col).astype(jnp.int32)
    oc = jnp.sum(le * u_row, axis=1, keepdims=True)
    cs = oc + jnp.minimum(k_pad, (i_col + 1) - oc)
    cs_d = jnp.sum(u_row * (q_row < _D)) + k_pad

    j_row = s_row
    out_small = jnp.sum((cs <= j_row).astype(jnp.int32), axis=0,
                        keepdims=True)

    b_row = u_row * (q_row >= _D).astype(jnp.int32)
    b_col = u_col * (q_col >= _D).astype(jnp.int32)
    r_col = cs_d + jnp.sum((q_row < q_col).astype(jnp.int32) * b_row,
                           axis=1, keepdims=True)
    hit = (r_col == j_row).astype(jnp.int32) * b_col
    out_big = jnp.sum(hit * q_col, axis=0, keepdims=True)

    picked = (jnp.where(j_row < cs_d, out_small, out_big)
              + (tgt_ref[0, 0] - _TGT))
    out_ref[...] = picked.reshape(_NPAD)[:_TGT]


def _run(attn2d, target_num, top_k):
    vals, idx = pl.pallas_call(
        _stage1_body,
        grid=(_S // _BLK,),
        in_specs=[pl.BlockSpec((_BLK, _T), lambda i: (i, 0))],
        out_specs=[pl.BlockSpec((_BLK, 1), lambda i: (i, 0)),
                   pl.BlockSpec((_BLK, 1), lambda i: (i, 0))],
        out_shape=[jax.ShapeDtypeStruct((_S, 1), jnp.float32),
                   jax.ShapeDtypeStruct((_S, 1), jnp.int32)],
    )(attn2d)

    topk_arr = jnp.asarray(top_k, jnp.int32).reshape(1, 1)
    tgt_arr = jnp.asarray(target_num, jnp.int32).reshape(1, 1)
    picked, vrow = pl.pallas_call(
        _stage2_body,
        out_shape=[jax.ShapeDtypeStruct((_TGT,), jnp.int32),
                   jax.ShapeDtypeStruct((1, _S), jnp.float32)],
    )(idx, vals, topk_arr, tgt_arr)
    return vrow.reshape(1, _S, 1), picked


def kernel(attn_qk, target_num, top_k):
    if attn_qk.ndim == 2:
        attn_qk = attn_qk[None]
    return _run(attn_qk.reshape(_S, _T), target_num, top_k)
